# Initial kernel scaffold; baseline (speedup 1.0000x reference)
import jax
import jax.numpy as jnp
from jax.experimental import pallas as pl


def kernel(pred_logits, pred_obj, pred_boxes, target_sizes):
    raise NotImplementedError("write your pallas kernel here")


# speedup vs baseline: 1.3285x; 1.3285x over previous
import jax
import jax.numpy as jnp
from jax import lax
from jax.experimental import pallas as pl
from jax.experimental.pallas import tpu as pltpu

_N = 5000
_C = 91
_NVALID = 81
_K = 100
_BIG = 2 ** 30


def _body(lgT_ref, lg_ref, obj_ref, boxes_ref, ts_ref,
          scores_ref, labels_ref, boxes_out_ref,
          selrows_ref, rowid_ref, bxrows_ref):
    lgT = lgT_ref[0]
    ob = obj_ref[0]
    eob = jnp.exp(-ob)
    c_iota = lax.broadcasted_iota(jnp.int32, (_C, _N), 0)
    pT = eob * jax.nn.sigmoid(lgT)
    pT = jnp.where(c_iota < _NVALID, pT, -1.0)
    nmax = jnp.max(pT, axis=0, keepdims=True)
    n_iota = lax.broadcasted_iota(jnp.int32, (1, _N), 1)
    lane_c = lax.broadcasted_iota(jnp.int32, (1, _C), 1)

    def rowstep(j, nm):
        m = jnp.max(nm)
        rid = jnp.min(jnp.where(nm == m, n_iota, _BIG))
        row_lg = lg_ref[0, pl.ds(rid, 1), :]
        e = jnp.max(jnp.where(n_iota == rid, eob, -jnp.inf))
        row_p = e * jax.nn.sigmoid(row_lg)
        row_p = jnp.where(lane_c < _NVALID, row_p, -1.0)
        selrows_ref[pl.ds(j, 1), :] = row_p
        rowid_ref[pl.ds(j, 1), :] = jnp.full((1, 1), rid, jnp.int32)
        return jnp.where(n_iota == rid, -1.0, nm)

    lax.fori_loop(0, _K, rowstep, nmax)

    S0 = selrows_ref[:, :]
    rid_col = rowid_ref[:, :]
    cK = lax.broadcasted_iota(jnp.int32, (_K, _C), 1)
    F = rid_col * jnp.int32(_C) + cK
    lane128 = lax.broadcasted_iota(jnp.int32, (1, 128), 1)
    sc0 = jnp.zeros((1, 128), jnp.float32)
    fl0 = jnp.zeros((1, 128), jnp.int32)

    def estep(q, carry):
        S, sc, fl_v = carry
        m = jnp.max(S)
        fl = jnp.min(jnp.where(S == m, F, _BIG))
        sc = jnp.where(lane128 == q, m, sc)
        fl_v = jnp.where(lane128 == q, fl, fl_v)
        rid_q = fl // jnp.int32(_C)
        bxrows_ref[pl.ds(q, 1), :] = boxes_ref[0, pl.ds(rid_q, 1), :]
        S = jnp.where(F == fl, jnp.float32(-1.0), S)
        return S, sc, fl_v

    _, sc, fl_v = lax.fori_loop(0, _K, estep, (S0, sc0, fl0))

    scores_ref[0] = sc[:, :_K]
    labels_ref[0] = (fl_v % jnp.int32(_C))[:, :_K]

    bx = bxrows_ref[:, :]
    cx = bx[:, 0:1]
    cy = bx[:, 1:2]
    w = bx[:, 2:3]
    h = bx[:, 3:4]
    xyxy = jnp.concatenate(
        [cx - 0.5 * w, cy - 0.5 * h, cx + 0.5 * w, cy + 0.5 * h], axis=1)
    tsv = ts_ref[0]
    scale = jnp.concatenate(
        [tsv[:, 1:2], tsv[:, 0:1], tsv[:, 1:2], tsv[:, 0:1]], axis=1)
    boxes_out_ref[0] = xyxy * scale


def kernel(pred_logits, pred_obj, pred_boxes, target_sizes):
    B, N, C = pred_logits.shape
    lgT = jnp.transpose(pred_logits, (0, 2, 1))
    obj3 = pred_obj[:, None, :]
    ts3 = target_sizes[:, None, :]

    scores3, labels3, boxes = pl.pallas_call(
        _body,
        grid=(B,),
        in_specs=[
            pl.BlockSpec((1, C, N), lambda b: (b, 0, 0)),
            pl.BlockSpec((1, N, C), lambda b: (b, 0, 0)),
            pl.BlockSpec((1, 1, N), lambda b: (b, 0, 0)),
            pl.BlockSpec((1, N, 4), lambda b: (b, 0, 0)),
            pl.BlockSpec((1, 1, 2), lambda b: (b, 0, 0)),
        ],
        out_specs=[
            pl.BlockSpec((1, 1, _K), lambda b: (b, 0, 0)),
            pl.BlockSpec((1, 1, _K), lambda b: (b, 0, 0)),
            pl.BlockSpec((1, _K, 4), lambda b: (b, 0, 0)),
        ],
        out_shape=[
            jax.ShapeDtypeStruct((B, 1, _K), jnp.float32),
            jax.ShapeDtypeStruct((B, 1, _K), jnp.int32),
            jax.ShapeDtypeStruct((B, _K, 4), jnp.float32),
        ],
        scratch_shapes=[
            pltpu.VMEM((_K, _C), jnp.float32),
            pltpu.VMEM((_K, 1), jnp.int32),
            pltpu.VMEM((_K, 4), jnp.float32),
        ],
        compiler_params=pltpu.CompilerParams(
            dimension_semantics=("arbitrary",)),
    )(lgT, pred_logits, obj3, pred_boxes, ts3)

    return scores3[:, 0, :], labels3[:, 0, :], boxes


# speedup vs baseline: 1.3472x; 1.0141x over previous
import jax
import jax.numpy as jnp
from jax import lax
from jax.experimental import pallas as pl
from jax.experimental.pallas import tpu as pltpu

_N = 5000
_C = 91
_NVALID = 81
_K = 100
_BIG = 2 ** 30


_NG = 40
_NL = 128


def _body(lgT_ref, lg_ref, obj_ref, boxes_ref, ts_ref,
          scores_ref, labels_ref, boxes_out_ref,
          selrows_ref, rowid_ref, bxrows_ref):
    lgT = lgT_ref[0]
    eob = jnp.exp(-obj_ref[0])
    c_iota = lax.broadcasted_iota(jnp.int32, (_C, _NG, _NL), 0)
    pT = eob * jax.nn.sigmoid(lgT)
    pT = jnp.where(c_iota < _NVALID, pT, -1.0)
    nmax = jnp.max(pT, axis=0)
    n_flat = (lax.broadcasted_iota(jnp.int32, (_NG, _NL), 0) * _NL
              + lax.broadcasted_iota(jnp.int32, (_NG, _NL), 1))
    nmax = jnp.where(n_flat < _N, nmax, -1.0)
    lane_c = lax.broadcasted_iota(jnp.int32, (1, _C), 1)

    def rowstep(j, nm):
        m = jnp.max(nm)
        rid = jnp.min(jnp.where(nm == m, n_flat, _BIG))
        row_lg = lg_ref[0, pl.ds(rid, 1), :]
        e = jnp.max(jnp.where(n_flat == rid, eob, -jnp.inf))
        row_p = e * jax.nn.sigmoid(row_lg)
        row_p = jnp.where(lane_c < _NVALID, row_p, -1.0)
        selrows_ref[pl.ds(j, 1), :] = row_p
        rowid_ref[pl.ds(j, 1), :] = jnp.full((1, 1), rid, jnp.int32)
        return jnp.where(n_flat == rid, -1.0, nm)

    lax.fori_loop(0, _K, rowstep, nmax)

    S0 = selrows_ref[:, :]
    rid_col = rowid_ref[:, :]
    cK = lax.broadcasted_iota(jnp.int32, (_K, _C), 1)
    F = rid_col * jnp.int32(_C) + cK
    lane128 = lax.broadcasted_iota(jnp.int32, (1, 128), 1)
    sc0 = jnp.zeros((1, 128), jnp.float32)
    fl0 = jnp.zeros((1, 128), jnp.int32)

    def estep(q, carry):
        S, sc, fl_v = carry
        m = jnp.max(S)
        fl = jnp.min(jnp.where(S == m, F, _BIG))
        sc = jnp.where(lane128 == q, m, sc)
        fl_v = jnp.where(lane128 == q, fl, fl_v)
        rid_q = fl // jnp.int32(_C)
        bxrows_ref[pl.ds(q, 1), :] = boxes_ref[0, pl.ds(rid_q, 1), :]
        S = jnp.where(F == fl, jnp.float32(-1.0), S)
        return S, sc, fl_v

    _, sc, fl_v = lax.fori_loop(0, _K, estep, (S0, sc0, fl0))

    scores_ref[0] = sc[:, :_K]
    labels_ref[0] = (fl_v % jnp.int32(_C))[:, :_K]

    bx = bxrows_ref[:, :]
    cx = bx[:, 0:1]
    cy = bx[:, 1:2]
    w = bx[:, 2:3]
    h = bx[:, 3:4]
    xyxy = jnp.concatenate(
        [cx - 0.5 * w, cy - 0.5 * h, cx + 0.5 * w, cy + 0.5 * h], axis=1)
    tsv = ts_ref[0]
    scale = jnp.concatenate(
        [tsv[:, 1:2], tsv[:, 0:1], tsv[:, 1:2], tsv[:, 0:1]], axis=1)
    boxes_out_ref[0] = xyxy * scale


def kernel(pred_logits, pred_obj, pred_boxes, target_sizes):
    B, N, C = pred_logits.shape
    npad = _NG * _NL - N
    lgT = jnp.transpose(pred_logits, (0, 2, 1))
    lgT4 = jnp.pad(lgT, ((0, 0), (0, 0), (0, npad))).reshape(B, C, _NG, _NL)
    obj4 = jnp.pad(pred_obj, ((0, 0), (0, npad))).reshape(B, _NG, _NL)
    ts3 = target_sizes[:, None, :]

    scores3, labels3, boxes = pl.pallas_call(
        _body,
        grid=(B,),
        in_specs=[
            pl.BlockSpec((1, C, _NG, _NL), lambda b: (b, 0, 0, 0)),
            pl.BlockSpec((1, N, C), lambda b: (b, 0, 0)),
            pl.BlockSpec((1, _NG, _NL), lambda b: (b, 0, 0)),
            pl.BlockSpec((1, N, 4), lambda b: (b, 0, 0)),
            pl.BlockSpec((1, 1, 2), lambda b: (b, 0, 0)),
        ],
        out_specs=[
            pl.BlockSpec((1, 1, _K), lambda b: (b, 0, 0)),
            pl.BlockSpec((1, 1, _K), lambda b: (b, 0, 0)),
            pl.BlockSpec((1, _K, 4), lambda b: (b, 0, 0)),
        ],
        out_shape=[
            jax.ShapeDtypeStruct((B, 1, _K), jnp.float32),
            jax.ShapeDtypeStruct((B, 1, _K), jnp.int32),
            jax.ShapeDtypeStruct((B, _K, 4), jnp.float32),
        ],
        scratch_shapes=[
            pltpu.VMEM((_K, _C), jnp.float32),
            pltpu.VMEM((_K, 1), jnp.int32),
            pltpu.VMEM((_K, 4), jnp.float32),
        ],
        compiler_params=pltpu.CompilerParams(
            dimension_semantics=("arbitrary",)),
    )(lgT4, pred_logits, obj4, pred_boxes, ts3)

    return scores3[:, 0, :], labels3[:, 0, :], boxes


# speedup vs baseline: 3.3388x; 2.4784x over previous
import jax
import jax.numpy as jnp
from jax import lax
from jax.experimental import pallas as pl
from jax.experimental.pallas import tpu as pltpu

_N = 5000
_C = 91
_NVALID = 81
_K = 100
_BIG = 2 ** 30
_NG = 40
_NL = 128


def _body(lgT_ref, lg_ref, obj_ref, boxes_ref, ts_ref,
          scores_ref, labels_ref, boxes_out_ref,
          selrows_ref, esel_ref, rowid_ref, bxrows_ref):
    B = lg_ref.shape[0]
    n_flat2 = (lax.broadcasted_iota(jnp.int32, (_NG, _NL), 0) * _NL
               + lax.broadcasted_iota(jnp.int32, (_NG, _NL), 1))
    n_flat3 = (lax.broadcasted_iota(jnp.int32, (B, _NG, _NL), 1) * _NL
               + lax.broadcasted_iota(jnp.int32, (B, _NG, _NL), 2))

    eob_list = [jnp.exp(-obj_ref[b]) for b in range(B)]

    def cstep(c, acc):
        lgc = lgT_ref[:, c]
        eob_all = acc[1]
        nm = jnp.maximum(acc[0], eob_all * jax.nn.sigmoid(lgc))
        return nm, eob_all

    eob_all = jnp.stack(eob_list)
    nm0 = eob_all * jax.nn.sigmoid(lgT_ref[:, 0])
    nmax, _ = lax.fori_loop(1, _NVALID, cstep, (nm0, eob_all))
    nmax = jnp.where(n_flat3 < _N, nmax, -1.0)

    def rowstep(j, M):
        mv = jnp.max(jnp.max(M, axis=2, keepdims=True), axis=1, keepdims=True)
        rv = jnp.min(jnp.min(jnp.where(M == mv, n_flat3, _BIG),
                             axis=2, keepdims=True), axis=1, keepdims=True)
        rowid_ref[:, pl.ds(j, 1), :] = rv
        for b in range(B):
            rid_b = jnp.max(rv[b])
            selrows_ref[b, pl.ds(j, 1), :] = lg_ref[b, pl.ds(rid_b, 1), :]
            e_b = jnp.max(jnp.where(n_flat2 == rid_b, eob_list[b], -jnp.inf))
            esel_ref[b, pl.ds(j, 1), :] = jnp.full((1, 1), e_b, jnp.float32)
        return jnp.where(n_flat3 == rv, -1.0, M)

    lax.fori_loop(0, _K, rowstep, nmax)

    SEL = selrows_ref[:, :, :]
    E = esel_ref[:, :, :]
    c3 = lax.broadcasted_iota(jnp.int32, (B, _K, _C), 2)
    P = E * jax.nn.sigmoid(SEL)
    P = jnp.where(c3 < _NVALID, P, -1.0)
    F = rowid_ref[:, :, :] * _C + c3
    lane3 = lax.broadcasted_iota(jnp.int32, (B, 1, 128), 2)
    sc0 = jnp.zeros((B, 1, 128), jnp.float32)
    fl0 = jnp.zeros((B, 1, 128), jnp.int32)

    def estep(q, carry):
        S, sc, fv = carry
        mv = jnp.max(jnp.max(S, axis=2, keepdims=True), axis=1, keepdims=True)
        fl = jnp.min(jnp.min(jnp.where(S == mv, F, _BIG),
                             axis=2, keepdims=True), axis=1, keepdims=True)
        sc = jnp.where(lane3 == q, mv, sc)
        fv = jnp.where(lane3 == q, fl, fv)
        for b in range(B):
            fb = jnp.max(fl[b])
            bxrows_ref[b, pl.ds(q, 1), :] = boxes_ref[b, pl.ds(fb // _C, 1), :]
        return jnp.where(F == fl, -1.0, S), sc, fv

    _, sc, fv = lax.fori_loop(0, _K, estep, (P, sc0, fl0))

    scores_ref[:, :, :] = sc[:, :, :_K]
    labels_ref[:, :, :] = (fv % _C)[:, :, :_K]

    bx = bxrows_ref[:, :, :]
    cx = bx[:, :, 0:1]
    cy = bx[:, :, 1:2]
    w = bx[:, :, 2:3]
    h = bx[:, :, 3:4]
    xyxy = jnp.concatenate(
        [cx - 0.5 * w, cy - 0.5 * h, cx + 0.5 * w, cy + 0.5 * h], axis=2)
    tsv = ts_ref[:, :, :]
    scale = jnp.concatenate(
        [tsv[:, :, 1:2], tsv[:, :, 0:1], tsv[:, :, 1:2], tsv[:, :, 0:1]],
        axis=2)
    boxes_out_ref[:, :, :] = xyxy * scale


def kernel(pred_logits, pred_obj, pred_boxes, target_sizes):
    B, N, C = pred_logits.shape
    npad = _NG * _NL - N
    lgT = jnp.transpose(pred_logits, (0, 2, 1))
    lgT4 = jnp.pad(lgT, ((0, 0), (0, 0), (0, npad))).reshape(B, C, _NG, _NL)
    obj4 = jnp.pad(pred_obj, ((0, 0), (0, npad))).reshape(B, _NG, _NL)
    ts3 = target_sizes[:, None, :]

    scores3, labels3, boxes = pl.pallas_call(
        _body,
        grid=(1,),
        in_specs=[
            pl.BlockSpec((B, C, _NG, _NL), lambda i: (0, 0, 0, 0)),
            pl.BlockSpec((B, N, C), lambda i: (0, 0, 0)),
            pl.BlockSpec((B, _NG, _NL), lambda i: (0, 0, 0)),
            pl.BlockSpec((B, N, 4), lambda i: (0, 0, 0)),
            pl.BlockSpec((B, 1, 2), lambda i: (0, 0, 0)),
        ],
        out_specs=[
            pl.BlockSpec((B, 1, _K), lambda i: (0, 0, 0)),
            pl.BlockSpec((B, 1, _K), lambda i: (0, 0, 0)),
            pl.BlockSpec((B, _K, 4), lambda i: (0, 0, 0)),
        ],
        out_shape=[
            jax.ShapeDtypeStruct((B, 1, _K), jnp.float32),
            jax.ShapeDtypeStruct((B, 1, _K), jnp.int32),
            jax.ShapeDtypeStruct((B, _K, 4), jnp.float32),
        ],
        scratch_shapes=[
            pltpu.VMEM((B, _K, _C), jnp.float32),
            pltpu.VMEM((B, _K, 1), jnp.float32),
            pltpu.VMEM((B, _K, 1), jnp.int32),
            pltpu.VMEM((B, _K, 4), jnp.float32),
        ],
        compiler_params=pltpu.CompilerParams(
            dimension_semantics=("arbitrary",)),
    )(lgT4, pred_logits, obj4, pred_boxes, ts3)

    return scores3[:, 0, :], labels3[:, 0, :], boxes
